# baseline (speedup 1.0000x reference)
import jax
import jax.numpy as jnp
from jax.experimental import pallas as pl

_ALPHA = 4
_IDX = (0, 4, 8, 12, 16, 21, 25, 29, 33, 37, 42, 46, 50, 54, 58, 63)


def _body(x_ref, slow_ref, fast_ref):
    v = x_ref[...]
    fast_ref[...] = v
    slow_ref[...] = v


def kernel(frames):
    C, T, H, W = frames.shape
    Ts = T // _ALPHA
    x = frames.reshape(C, T, 392, 128)

    def in_map(c, t):
        return (c, t, 0, 0)

    def slow_map(c, t):
        return (c, (15 * t + 62) // 63, 0, 0)

    slow, fast = pl.pallas_call(
        _body,
        grid=(C, T),
        in_specs=[pl.BlockSpec((1, 1, 392, 128), in_map)],
        out_specs=[
            pl.BlockSpec((1, 1, 392, 128), slow_map),
            pl.BlockSpec((1, 1, 392, 128), in_map),
        ],
        out_shape=[
            jax.ShapeDtypeStruct((C, Ts, 392, 128), frames.dtype),
            jax.ShapeDtypeStruct((C, T, 392, 128), frames.dtype),
        ],
    )(x)
    return (slow.reshape(C, Ts, H, W), fast.reshape(C, T, H, W))
